# Initial kernel scaffold; baseline (speedup 1.0000x reference)
import jax
import jax.numpy as jnp
from jax.experimental import pallas as pl


def kernel(R_ij, i, j, species, structures, centers, W):
    raise NotImplementedError("write your pallas kernel here")


# speedup vs baseline: 3.4386x; 3.4386x over previous
import functools

import jax
import jax.numpy as jnp
import numpy as np
from jax.experimental import pallas as pl
from jax.experimental.pallas import tpu as pltpu

_CUTOFF = 5.0
_N_SPECIES = 4
_BESSEL_ZEROS = [
    np.array([3.141592653589793, 6.283185307179586, 9.42477796076938,
              12.566370614359172, 15.707963267948966, 18.84955592153876,
              21.991148575128552, 25.132741228718345]),
    np.array([4.493409457909064, 7.725251836937707, 10.904121659428899,
              14.066193912831473, 17.220755271930768, 20.371302959287563,
              23.519452498689006]),
    np.array([5.76345919689455, 9.095011330476355, 12.322940970566582,
              15.514603010886749, 18.689036355362822, 21.853874222709714]),
]
_NL = [8, 7, 6]
_MOFF = [0, 8, 29]
_MS = [8, 21, 30]
_F = 64

_EDGE_BLK = 640
_WIN_A = _EDGE_BLK + 8
_KEY_BLK = 1024
_WIN_B = _KEY_BLK + 8


def _bessel(l, x):
    x = jnp.maximum(x, 1e-8)
    if l == 0:
        return jnp.sin(x) / x
    if l == 1:
        return jnp.sin(x) / x**2 - jnp.cos(x) / x
    return (3.0 / x**3 - 1.0 / x) * jnp.sin(x) - 3.0 * jnp.cos(x) / x**2


def _accum_kernel(r0s_ref, rank_ref, rs_ref, out_ref):
    b = pl.program_id(0)

    @pl.when(b == 0)
    def _init():
        out_ref[...] = jnp.zeros_like(out_ref)

    blk = rs_ref.shape[0]
    R = rs_ref[...]
    x = R[:, 0:1]
    y = R[:, 1:2]
    z = R[:, 2:3]
    r = jnp.sqrt(x * x + y * y + z * z)
    r_safe = jnp.maximum(r, 1e-8)
    inv = 1.0 / r_safe
    xh, yh, zh = x * inv, y * inv, z * inv
    mask = (r < _CUTOFF).astype(jnp.float32)

    c1 = 0.4886025119029199
    c2 = 1.0925484305920792
    ys = [
        [jnp.full_like(x, 0.28209479177387814)],
        [c1 * yh, c1 * zh, c1 * xh],
        [c2 * xh * yh, c2 * yh * zh,
         0.31539156525252005 * (3.0 * zh * zh - 1.0),
         c2 * xh * zh, 0.5462742152960396 * (xh * xh - yh * yh)],
    ]
    pieces = []
    for l in range(3):
        rnl = jnp.concatenate(
            [_bessel(l, float(zz) * r_safe / _CUTOFF) for zz in _BESSEL_ZEROS[l]],
            axis=1) * mask
        for ylm in ys[l]:
            pieces.append(rnl * ylm)
    pieces.append(jnp.zeros((blk, _F - 59), dtype=jnp.float32))
    u = jnp.concatenate(pieces, axis=1)

    r0 = r0s_ref[b]
    r0a = (r0 // 8) * 8
    lr = rank_ref[...] - r0a
    iota = jax.lax.broadcasted_iota(jnp.int32, (blk, _WIN_A), 1)
    oh = (lr == iota).astype(jnp.float32)
    contrib = jax.lax.dot_general(
        oh, u, (((0,), (0,)), ((), ())),
        preferred_element_type=jnp.float32)
    cur = out_ref[pl.ds(r0a, _WIN_A), :]
    out_ref[pl.ds(r0a, _WIN_A), :] = cur + contrib


def _expand_kernel(rk0s_ref, rok_ref, occ_ref, w_ref, u_ref, out_ref):
    q = pl.program_id(0)
    rk0 = rk0s_ref[q]
    rk0a = (rk0 // 8) * 8
    uw = u_ref[pl.ds(rk0a, _WIN_B), :]
    nodes = rok_ref.shape[0]
    iota = jax.lax.broadcasted_iota(jnp.int32, (nodes, _WIN_B), 1)
    uks = []
    for sj in range(_N_SPECIES):
        lr = rok_ref[:, sj:sj + 1] - rk0a
        occ = occ_ref[:, sj:sj + 1]
        oh = jnp.where((lr == iota) & (occ > 0), 1.0, 0.0)
        uks.append(jax.lax.dot_general(
            oh, uw, (((1,), (0,)), ((), ())),
            preferred_element_type=jnp.float32))
    outs = []
    for s in range(_N_SPECIES):
        acc = uks[0] * w_ref[0, s]
        for sj in range(1, _N_SPECIES):
            acc = acc + uks[sj] * w_ref[sj, s]
        outs.append(acc)
    out_ref[...] = jnp.concatenate(outs, axis=1)


@jax.jit
def kernel(R_ij, i, j, species, structures, centers, W):
    n_nodes = species.shape[0]
    n_edges = i.shape[0]
    S = _N_SPECIES
    ks = ((n_nodes * S + _KEY_BLK - 1) // _KEY_BLK) * _KEY_BLK
    ur_size = ks + _KEY_BLK

    perm = jnp.argsort(species, stable=True)
    node_pos = jnp.zeros((n_nodes,), jnp.int32).at[perm].set(
        jnp.arange(n_nodes, dtype=jnp.int32))
    key = node_pos[i] * S + species[j]
    order = jnp.argsort(key)
    keys = key[order]
    rs = R_ij[order]

    pad = (-n_edges) % _EDGE_BLK
    if pad:
        keys = jnp.concatenate([keys, jnp.full((pad,), ks - 1, keys.dtype)])
        rs = jnp.concatenate(
            [rs, jnp.full((pad, 3), 2.0 * _CUTOFF, rs.dtype)])
    ep = n_edges + pad

    nb = jnp.concatenate([
        jnp.zeros((1,), jnp.int32),
        (keys[1:] != keys[:-1]).astype(jnp.int32)])
    rank = jnp.cumsum(nb).astype(jnp.int32)
    r0s = rank[::_EDGE_BLK]

    karr = jnp.arange(ks, dtype=keys.dtype)
    pos = jnp.searchsorted(keys, karr)
    posc = jnp.minimum(pos, ep - 1)
    occ = ((pos < ep) & (keys[posc] == karr)).astype(jnp.int32)
    rok = rank[posc]
    rk0s = rok[::_KEY_BLK]
    rok4 = rok.reshape(ks // S, S)
    occ4 = occ.reshape(ks // S, S)

    grid_a = ep // _EDGE_BLK
    u_acc = pl.pallas_call(
        _accum_kernel,
        grid_spec=pltpu.PrefetchScalarGridSpec(
            num_scalar_prefetch=1,
            grid=(grid_a,),
            in_specs=[
                pl.BlockSpec((_EDGE_BLK, 1), lambda b, r0s: (b, 0)),
                pl.BlockSpec((_EDGE_BLK, 3), lambda b, r0s: (b, 0)),
            ],
            out_specs=pl.BlockSpec((ur_size, _F), lambda b, r0s: (0, 0)),
        ),
        out_shape=jax.ShapeDtypeStruct((ur_size, _F), jnp.float32),
    )(r0s, rank[:, None], rs)

    grid_b = ks // _KEY_BLK
    nodes_blk = _KEY_BLK // S
    feat = pl.pallas_call(
        _expand_kernel,
        grid_spec=pltpu.PrefetchScalarGridSpec(
            num_scalar_prefetch=1,
            grid=(grid_b,),
            in_specs=[
                pl.BlockSpec((nodes_blk, S), lambda q, rk0s: (q, 0)),
                pl.BlockSpec((nodes_blk, S), lambda q, rk0s: (q, 0)),
                pl.BlockSpec(memory_space=pltpu.SMEM),
                pl.BlockSpec((ur_size, _F), lambda q, rk0s: (0, 0)),
            ],
            out_specs=pl.BlockSpec((nodes_blk, S * _F),
                                   lambda q, rk0s: (q, 0)),
        ),
        out_shape=jax.ShapeDtypeStruct((ks // S, S * _F), jnp.float32),
    )(rk0s, rok4, occ4, W, u_acc)

    feat = feat[:n_nodes].reshape(n_nodes, S, _F)
    counts = jnp.zeros((S,), jnp.int32).at[species].add(1)
    starts = jnp.concatenate(
        [jnp.zeros((1,), jnp.int32), jnp.cumsum(counts)[:-1]])
    species_sorted = species[perm]
    ranks_c = jnp.arange(n_nodes, dtype=jnp.int32) - starts[species_sorted]
    blocks = []
    for l in range(3):
        M = _MS[l]
        D = feat[:, :, _MOFF[l]:_MOFF[l] + M]
        base = starts[species_sorted] * (S * M) + ranks_c * M
        c_k = counts[species_sorted]
        idx = (base[:, None, None]
               + jnp.arange(S)[None, :, None] * c_k[:, None, None] * M
               + jnp.arange(M)[None, None, :])
        out_l = jnp.zeros((n_nodes * S * M,), jnp.float32).at[
            idx.reshape(-1)].set(D.reshape(-1))
        blocks.append(out_l)
    return jnp.concatenate(blocks)
